# baseline (speedup 1.0000x reference)
import functools

import jax
import jax.numpy as jnp
from jax import lax
from jax.experimental import pallas as pl
from jax.experimental.pallas import tpu as pltpu
from jax.experimental.pallas import tpu_sc as plsc

F = 26
B = 16384
V = 100000
D = 16

NC = 2
NS = 16
NW = NC * NS
SLICES = F * D
PER_W = SLICES // NW
BC = 4096
NB = B // BC
UNROLL = 8

_mesh = plsc.VectorSubcoreMesh(core_axis_name="c", subcore_axis_name="s")


@functools.partial(
    pl.kernel,
    mesh=_mesh,
    compiler_params=pltpu.CompilerParams(
        use_tc_tiling_on_sc=True, needs_layout_passes=False
    ),
    out_type=jax.ShapeDtypeStruct((F * D, B), jnp.float32),
    scratch_types=[
        pltpu.VMEM((V,), jnp.float32),
        pltpu.VMEM((2, BC), jnp.int32),
        pltpu.VMEM((2, BC), jnp.float32),
        pltpu.SemaphoreType.DMA,
        pltpu.SemaphoreType.DMA,
        pltpu.SemaphoreType.DMA,
        pltpu.SemaphoreType.DMA,
        pltpu.SemaphoreType.DMA,
    ],
)
def _slice_gather(
    ids_hbm, tabt_hbm, out_hbm, col_v, idx_v, row_v,
    sem_col, sem_i0, sem_i1, sem_o0, sem_o1,
):
    wid = lax.axis_index("s") * NC + lax.axis_index("c")
    sem_i = (sem_i0, sem_i1)
    sem_o = (sem_o0, sem_o1)

    def slice_body(j, carry):
        s = j * NW + wid
        f = s // D
        d = s % D
        col_dma = pltpu.async_copy(tabt_hbm.at[f, d], col_v, sem_col)
        ids_dma = pltpu.async_copy(
            ids_hbm.at[f, pl.ds(0, BC)], idx_v.at[0], sem_i[0]
        )
        col_dma.wait()

        for cb in range(NB):
            b = cb % 2
            if cb + 1 < NB:
                nxt = pltpu.async_copy(
                    ids_hbm.at[f, pl.ds((cb + 1) * BC, BC)],
                    idx_v.at[(cb + 1) % 2],
                    sem_i[(cb + 1) % 2],
                )
            ids_dma.wait()
            if cb + 1 < NB:
                ids_dma = nxt

            drain = pltpu.make_async_copy(
                row_v.at[b], out_hbm.at[s, pl.ds(cb * BC, BC)], sem_o[b]
            )
            if cb >= 2:
                drain.wait()
            else:
                @pl.when(j > 0)
                def _():
                    drain.wait()

            @plsc.parallel_loop(0, BC // 16, unroll=UNROLL)
            def _(i, _b=b):
                off = i * 16
                idx = idx_v[_b, pl.ds(off, 16)]
                row_v[_b, pl.ds(off, 16)] = plsc.load_gather(col_v, [idx])

            pltpu.async_copy(
                row_v.at[b], out_hbm.at[s, pl.ds(cb * BC, BC)], sem_o[b]
            )
        return carry

    lax.fori_loop(0, PER_W, slice_body, 0)

    last = NW * PER_W - 1
    for b in range(2):
        pltpu.make_async_copy(
            row_v.at[b], out_hbm.at[last, pl.ds(b * BC, BC)], sem_o[b]
        ).wait()


def kernel(ids, tables):
    tabt = jnp.transpose(tables, (0, 2, 1))
    out_t = _slice_gather(ids, tabt)
    return out_t.T


# speedup vs baseline: 1.0269x; 1.0269x over previous
import functools

import jax
import jax.numpy as jnp
from jax import lax
from jax.experimental import pallas as pl
from jax.experimental.pallas import tpu as pltpu
from jax.experimental.pallas import tpu_sc as plsc

F = 26
B = 16384
V = 100000
D = 16

NC = 2
NS = 16
NW = NC * NS
SLICES = F * D
PER_W = SLICES // NW
BC = 4096
NB = B // BC
UNROLL = 8

_mesh = plsc.VectorSubcoreMesh(core_axis_name="c", subcore_axis_name="s")


@functools.partial(
    pl.kernel,
    mesh=_mesh,
    compiler_params=pltpu.CompilerParams(
        use_tc_tiling_on_sc=True, needs_layout_passes=False
    ),
    out_type=jax.ShapeDtypeStruct((F * D, B), jnp.float32),
    scratch_types=[
        pltpu.VMEM((V,), jnp.float32),
        pltpu.VMEM((2, BC), jnp.int32),
        pltpu.VMEM((2, BC), jnp.float32),
        pltpu.SemaphoreType.DMA,
        pltpu.SemaphoreType.DMA,
        pltpu.SemaphoreType.DMA,
        pltpu.SemaphoreType.DMA,
        pltpu.SemaphoreType.DMA,
    ],
)
def _slice_gather(
    ids_hbm, tabt_hbm, out_hbm, col_v, idx_v, row_v,
    sem_col, sem_i0, sem_i1, sem_o0, sem_o1,
):
    wid = lax.axis_index("s") * NC + lax.axis_index("c")
    sem_i = (sem_i0, sem_i1)
    sem_o = (sem_o0, sem_o1)

    def slice_body(j, carry):
        s = wid * PER_W + j
        f = s // D
        d = s % D
        col_dma = pltpu.async_copy(tabt_hbm.at[f, d], col_v, sem_col)
        ids_dma = pltpu.async_copy(
            ids_hbm.at[f, pl.ds(0, BC)], idx_v.at[0], sem_i[0]
        )
        col_dma.wait()

        for cb in range(NB):
            b = cb % 2
            if cb + 1 < NB:
                nxt = pltpu.async_copy(
                    ids_hbm.at[f, pl.ds((cb + 1) * BC, BC)],
                    idx_v.at[(cb + 1) % 2],
                    sem_i[(cb + 1) % 2],
                )
            ids_dma.wait()
            if cb + 1 < NB:
                ids_dma = nxt

            drain = pltpu.make_async_copy(
                row_v.at[b], out_hbm.at[s, pl.ds(cb * BC, BC)], sem_o[b]
            )
            if cb >= 2:
                drain.wait()
            else:
                @pl.when(j > 0)
                def _():
                    drain.wait()

            @plsc.parallel_loop(0, BC // 16, unroll=UNROLL)
            def _(i, _b=b):
                off = i * 16
                idx = idx_v[_b, pl.ds(off, 16)]
                row_v[_b, pl.ds(off, 16)] = plsc.load_gather(col_v, [idx])

            pltpu.async_copy(
                row_v.at[b], out_hbm.at[s, pl.ds(cb * BC, BC)], sem_o[b]
            )
        return carry

    lax.fori_loop(0, PER_W, slice_body, 0)

    last = NW * PER_W - 1
    for b in range(2):
        pltpu.make_async_copy(
            row_v.at[b], out_hbm.at[last, pl.ds(b * BC, BC)], sem_o[b]
        ).wait()


def kernel(ids, tables):
    tabt = jnp.transpose(tables, (0, 2, 1))
    out_t = _slice_gather(ids, tabt)
    return out_t.T
